# baseline (speedup 1.0000x reference)
import functools

import jax
import jax.numpy as jnp
from jax import lax
from jax.experimental import pallas as pl
from jax.experimental.pallas import tpu as pltpu
from jax.experimental.pallas import tpu_sc as plsc

B, L, E = 4096, 200, 64
N = B * L
NC, NS = 2, 16
NW = NC * NS
W = N // NW
CH = 128
NCH = W // CH
M = 8
K = 4
NG = NCH // M

_mesh = plsc.VectorSubcoreMesh(core_axis_name="c", subcore_axis_name="s")


@functools.partial(
    pl.kernel,
    out_type=jax.ShapeDtypeStruct((N, E), jnp.float32),
    mesh=_mesh,
    scratch_types=[
        pltpu.VMEM((NCH, CH), jnp.int32),
        pltpu.VMEM((M, CH, E), jnp.float32),
        [pltpu.SemaphoreType.DMA] * M,
        [pltpu.SemaphoreType.DMA] * M,
    ],
    compiler_params=pltpu.CompilerParams(use_tc_tiling_on_sc=False),
)
def _emb(idx_hbm, table_hbm, out_hbm, idx_v, rows_v, gsems, ssems):
    wid = lax.axis_index("s") * NC + lax.axis_index("c")
    base_ch = wid * NCH
    pltpu.sync_copy(idx_hbm.at[pl.ds(base_ch, NCH)], idx_v)

    def gather(j, b):
        return pltpu.make_async_copy(
            table_hbm.at[idx_v.at[j]], rows_v.at[b], gsems[b]
        )

    def store(j, b):
        return pltpu.make_async_copy(
            rows_v.at[b], out_hbm.at[pl.ds((base_ch + j) * CH, CH)], ssems[b]
        )

    for b in range(K):
        gather(b, b).start()

    def body(g, carry):
        for b in range(M):
            j = g * M + b
            gather(j, b).wait()
            store(j, b).start()
            bn = (b + K) % M
            @pl.when(j >= K)
            def _():
                store(j - K, bn).wait()

            @pl.when(j + K < NCH)
            def _():
                gather(j + K, bn).start()
        return carry

    lax.fori_loop(0, NG, body, 0)

    for b in range(K):
        j = NCH - K + b
        store(j, j % M).wait()


def kernel(tok_ids, table):
    idx = tok_ids.reshape(NW * NCH, CH).astype(jnp.int32)
    out = _emb(idx, table)
    return out.reshape(B, L, E)
